# baseline (speedup 1.0000x reference)
import jax
import jax.numpy as jnp
from jax.experimental import pallas as pl
from jax.experimental.pallas import tpu as pltpu

_N = 10


def _body(pk_v, wc_v, wa_v, out1_ref, out2_ref):
    a = pk_v[...]
    a1 = a[0:10, 0:195]
    st = a[0:10, 64:195]
    nb = jax.lax.bitcast_convert_type(a[0:10, 240:243], jnp.int32)
    bc = a[10:11, :]
    ba = a[11:12, :]

    out1 = jax.lax.dot_general(a1, wc_v[...],
                               (((1,), (0,)), ((), ())),
                               preferred_element_type=jnp.float32)
    out1_ref[...] = out1 + bc

    col = jax.lax.broadcasted_iota(jnp.int32, (_N, _N), 1)
    row = jax.lax.broadcasted_iota(jnp.int32, (_N, _N), 0)
    cnt = (row == col).astype(jnp.float32)
    for k in range(3):
        cnt += (nb[:, k:k + 1] == col).astype(jnp.float32)
    m = cnt * 0.25

    vec4 = jax.lax.dot_general(m, st, (((1,), (0,)), ((), ())),
                               preferred_element_type=jnp.float32)
    out2 = jax.lax.dot_general(vec4, wa_v[...],
                               (((1,), (0,)), ((), ())),
                               preferred_element_type=jnp.float32)
    out2_ref[...] = out2 + ba


@jax.jit
def kernel(spatial, structural, neighbour, W_comb, b_comb, W_agg, b_agg):
    nb_f = jax.lax.bitcast_convert_type(neighbour.astype(jnp.int32),
                                        jnp.float32)
    pack = jnp.zeros((16, 256), jnp.float32)
    pack = pack.at[0:10, 0:64].set(spatial)
    pack = pack.at[0:10, 64:195].set(structural)
    pack = pack.at[0:10, 240:243].set(nb_f)
    pack = pack.at[10, :].set(b_comb)
    pack = pack.at[11, :].set(b_agg)

    out_shape = (jax.ShapeDtypeStruct((_N, 256), jnp.float32),
                 jax.ShapeDtypeStruct((_N, 256), jnp.float32))
    vmem_spec = pl.BlockSpec(memory_space=pltpu.VMEM)
    return pl.pallas_call(
        _body,
        out_shape=out_shape,
        in_specs=[vmem_spec] * 3,
    )(pack, W_comb.T, W_agg.T)
